# baseline (speedup 1.0000x reference)
import functools

import jax
import jax.numpy as jnp
from jax import lax
from jax.experimental import pallas as pl
from jax.experimental.pallas import tpu as pltpu
from jax.experimental.pallas import tpu_sc as plsc

PROMPT_LENGTH = 20
EMBED_DIM = 64
BATCH = 4096
SEQ_LEN = 200

NUM_CORES = 2
NUM_SUBCORES = 16
NUM_WORKERS = NUM_CORES * NUM_SUBCORES
LANES = 16

ROWS = BATCH * SEQ_LEN
BATCH_PER_WORKER = BATCH // NUM_WORKERS
ROWS_PER_WORKER = ROWS // NUM_WORKERS
CHUNK_BATCHES = 2
CHUNK_ROWS = CHUNK_BATCHES * SEQ_LEN
CHUNKS_PER_WORKER = ROWS_PER_WORKER // CHUNK_ROWS
GROUPS_PER_CHUNK = CHUNK_ROWS // LANES
PERIOD_ROWS = 400
GROUP_PERIOD = PERIOD_ROWS // LANES
TAB_WORDS = PROMPT_LENGTH * EMBED_DIM


@functools.partial(
    pl.kernel,
    mesh=plsc.VectorSubcoreMesh(core_axis_name="c", subcore_axis_name="s"),
    out_type=jax.ShapeDtypeStruct((BATCH, SEQ_LEN, EMBED_DIM), jnp.float32),
    compiler_params=pltpu.CompilerParams(use_tc_tiling_on_sc=True,
                                         needs_layout_passes=False),
    scratch_types=[
        pltpu.VMEM((2 * TAB_WORDS,), jnp.float32),
        pltpu.VMEM((ROWS_PER_WORKER,), jnp.int32),
        pltpu.VMEM((CHUNK_BATCHES, SEQ_LEN, EMBED_DIM), jnp.float32),
        pltpu.VMEM((CHUNK_BATCHES, SEQ_LEN, EMBED_DIM), jnp.float32),
        pltpu.SemaphoreType.DMA,
        pltpu.SemaphoreType.DMA,
        pltpu.SemaphoreType.DMA,
    ],
)
def _embed(idx_hbm, p_hbm, n_hbm, out_hbm,
           tab_v, adj_v, rows0, rows1, sem_in, sem0, sem1):
    wid = lax.axis_index("s") * NUM_CORES + lax.axis_index("c")
    row_bufs = (rows0, rows1)
    sems = (sem0, sem1)
    w0 = wid * ROWS_PER_WORKER

    idx_cp = pltpu.async_copy(idx_hbm.at[pl.ds(w0, ROWS_PER_WORKER)],
                              adj_v, sem_in)
    pltpu.sync_copy(p_hbm, tab_v.at[pl.ds(0, TAB_WORDS)])
    pltpu.sync_copy(n_hbm, tab_v.at[pl.ds(TAB_WORDS, TAB_WORDS)])
    idx_cp.wait()

    lane = lax.iota(jnp.int32, LANES)
    ntab = PROMPT_LENGTH * EMBED_DIM
    sel = {
        "zero": lane * 0,
        "norm": lane * 0 + ntab,
        "m8": jnp.where(lane < 12, 0, ntab),
        "m16": jnp.where(lane < 4, 0, ntab),
        "m192": jnp.where(lane < 8, ntab, 0),
    }

    def col_off(j):
        pos = [(j * LANES + l) % SEQ_LEN for l in range(LANES)]
        key = [PROMPT_LENGTH * EMBED_DIM if p >= PROMPT_LENGTH else 0
               for p in pos]
        for name, vec in sel.items():
            ref = {"zero": [0] * LANES,
                   "norm": [ntab] * LANES,
                   "m8": [0 if l < 12 else ntab for l in range(LANES)],
                   "m16": [0 if l < 4 else ntab for l in range(LANES)],
                   "m192": [ntab if l < 8 else 0 for l in range(LANES)],
                   }[name]
            if key == ref:
                return vec
        raise AssertionError(f"unhandled group phase {j}")

    def adj_body(p, carry):
        for j in range(GROUP_PERIOD):
            sl = pl.ds(p * PERIOD_ROWS + j * LANES, LANES)
            adj_v[sl] = adj_v[sl] * EMBED_DIM + col_off(j)
        return carry

    lax.fori_loop(0, ROWS_PER_WORKER // PERIOD_ROWS, adj_body, 0)

    def expand_chunk(ci, rows_v):
        def dest(r0, i):
            r = r0 + i // 4
            return (r // SEQ_LEN, lax.rem(r, SEQ_LEN),
                    pl.ds((i % 4) * LANES, LANES))

        def store8(pend):
            r0, vals = pend
            for i, v in enumerate(vals):
                b2, s2, ksl = dest(r0, i)
                rows_v[b2, s2, ksl] = v

        def group_body(g, carry):
            av = adj_v[pl.ds(ci * CHUNK_ROWS + g * LANES, LANES)]
            base = g * LANES
            pend = None
            for l0 in range(0, LANES, 2):
                s0, s1 = av[l0], av[l0 + 1]
                loads = []
                for i in range(8):
                    s = s0 if i < 4 else s1
                    k = (i % 4) * LANES
                    loads.append(tab_v[pl.ds(s + k, LANES)])
                    if pend is not None:
                        r0, vals = pend
                        b2, s2, ksl = dest(r0, i)
                        rows_v[b2, s2, ksl] = vals[i]
                pend = (base + l0, loads)
            store8(pend)
            return carry

        lax.fori_loop(0, GROUPS_PER_CHUNK, group_body, 0)

    def out_slice(ci):
        return out_hbm.at[pl.ds(wid * BATCH_PER_WORKER + ci * CHUNK_BATCHES,
                                CHUNK_BATCHES)]

    def start_out(ci, b):
        pltpu.async_copy(row_bufs[b], out_slice(ci), sems[b])

    def wait_out(ci, b):
        pltpu.make_async_copy(row_bufs[b], out_slice(ci), sems[b]).wait()

    expand_chunk(0, rows0)
    start_out(0, 0)
    expand_chunk(1, rows1)
    start_out(1, 1)

    def step_body(stp, carry):
        for b in range(2):
            ci = stp * 2 + b
            wait_out(ci - 2, b)
            expand_chunk(ci, row_bufs[b])
            start_out(ci, b)
        return carry

    lax.fori_loop(1, CHUNKS_PER_WORKER // 2, step_body, 0)
    wait_out(CHUNKS_PER_WORKER - 2, 0)
    wait_out(CHUNKS_PER_WORKER - 1, 1)


def kernel(input, prompt_table, normal_table):
    idx = jnp.maximum(input.astype(jnp.int32), 0).reshape(ROWS)
    return _embed(idx,
                  prompt_table.reshape(-1),
                  normal_table[:PROMPT_LENGTH].reshape(-1))


# speedup vs baseline: 1.2896x; 1.2896x over previous
import functools

import jax
import jax.numpy as jnp
from jax import lax
from jax.experimental import pallas as pl
from jax.experimental.pallas import tpu as pltpu
from jax.experimental.pallas import tpu_sc as plsc

PROMPT_LENGTH = 20
EMBED_DIM = 64
BATCH = 4096
SEQ_LEN = 200

NUM_CORES = 2
NUM_SUBCORES = 16
NUM_WORKERS = NUM_CORES * NUM_SUBCORES
LANES = 16

ROWS = BATCH * SEQ_LEN
BATCH_PER_WORKER = BATCH // NUM_WORKERS
ROWS_PER_WORKER = ROWS // NUM_WORKERS
CHUNK_BATCHES = 2
CHUNK_ROWS = CHUNK_BATCHES * SEQ_LEN
CHUNKS_PER_WORKER = ROWS_PER_WORKER // CHUNK_ROWS
GROUPS_PER_CHUNK = CHUNK_ROWS // LANES
PERIOD_ROWS = 400
GROUP_PERIOD = PERIOD_ROWS // LANES
TAB_WORDS = PROMPT_LENGTH * EMBED_DIM


@functools.partial(
    pl.kernel,
    mesh=plsc.VectorSubcoreMesh(core_axis_name="c", subcore_axis_name="s"),
    out_type=jax.ShapeDtypeStruct((ROWS, EMBED_DIM), jnp.float32),
    compiler_params=pltpu.CompilerParams(use_tc_tiling_on_sc=True,
                                         needs_layout_passes=False),
    scratch_types=[
        pltpu.VMEM((2 * TAB_WORDS,), jnp.float32),
        pltpu.VMEM((ROWS_PER_WORKER,), jnp.int32),
        pltpu.VMEM((CHUNK_ROWS, EMBED_DIM), jnp.float32),
        pltpu.VMEM((CHUNK_ROWS, EMBED_DIM), jnp.float32),
        pltpu.SemaphoreType.DMA,
        pltpu.SemaphoreType.DMA,
        pltpu.SemaphoreType.DMA,
    ],
)
def _embed(idx_hbm, p_hbm, n_hbm, out_hbm,
           tab_v, adj_v, rows0, rows1, sem_in, sem0, sem1):
    wid = lax.axis_index("s") * NUM_CORES + lax.axis_index("c")
    row_bufs = (rows0, rows1)
    sems = (sem0, sem1)
    w0 = wid * ROWS_PER_WORKER

    idx_cp = pltpu.async_copy(idx_hbm.at[pl.ds(w0, ROWS_PER_WORKER)],
                              adj_v, sem_in)
    pltpu.sync_copy(p_hbm, tab_v.at[pl.ds(0, TAB_WORDS)])
    pltpu.sync_copy(n_hbm, tab_v.at[pl.ds(TAB_WORDS, TAB_WORDS)])
    idx_cp.wait()

    lane = lax.iota(jnp.int32, LANES)
    ntab = PROMPT_LENGTH * EMBED_DIM
    sel = {
        "zero": lane * 0,
        "norm": lane * 0 + ntab,
        "m8": jnp.where(lane < 12, 0, ntab),
        "m16": jnp.where(lane < 4, 0, ntab),
        "m192": jnp.where(lane < 8, ntab, 0),
    }

    def col_off(j):
        pos = [(j * LANES + l) % SEQ_LEN for l in range(LANES)]
        key = [PROMPT_LENGTH * EMBED_DIM if p >= PROMPT_LENGTH else 0
               for p in pos]
        for name, vec in sel.items():
            ref = {"zero": [0] * LANES,
                   "norm": [ntab] * LANES,
                   "m8": [0 if l < 12 else ntab for l in range(LANES)],
                   "m16": [0 if l < 4 else ntab for l in range(LANES)],
                   "m192": [ntab if l < 8 else 0 for l in range(LANES)],
                   }[name]
            if key == ref:
                return vec
        raise AssertionError(f"unhandled group phase {j}")

    def adj_body(p, carry):
        for j in range(GROUP_PERIOD):
            sl = pl.ds(p * PERIOD_ROWS + j * LANES, LANES)
            adj_v[sl] = adj_v[sl] * EMBED_DIM + col_off(j)
        return carry

    lax.fori_loop(0, ROWS_PER_WORKER // PERIOD_ROWS, adj_body, 0)

    def expand_chunk(ci, rows_v):
        def store8(pend):
            r0, vals = pend
            for i, v in enumerate(vals):
                rows_v[r0 + i // 4, pl.ds((i % 4) * LANES, LANES)] = v

        def group_body(g, carry):
            av = adj_v[pl.ds(ci * CHUNK_ROWS + g * LANES, LANES)]
            base = g * LANES
            pend = None
            for l0 in range(0, LANES, 2):
                s0, s1 = av[l0], av[l0 + 1]
                loads = []
                for i in range(8):
                    s = s0 if i < 4 else s1
                    k = (i % 4) * LANES
                    loads.append(tab_v[pl.ds(s + k, LANES)])
                    if pend is not None:
                        r0, vals = pend
                        rows_v[r0 + i // 4,
                               pl.ds((i % 4) * LANES, LANES)] = vals[i]
                pend = (base + l0, loads)
            store8(pend)
            return carry

        lax.fori_loop(0, GROUPS_PER_CHUNK, group_body, 0)

    def out_slice(ci):
        return out_hbm.at[pl.ds(w0 + ci * CHUNK_ROWS, CHUNK_ROWS)]

    def start_out(ci, b):
        pltpu.async_copy(row_bufs[b], out_slice(ci), sems[b])

    def wait_out(ci, b):
        pltpu.make_async_copy(row_bufs[b], out_slice(ci), sems[b]).wait()

    expand_chunk(0, rows0)
    start_out(0, 0)
    expand_chunk(1, rows1)
    start_out(1, 1)

    def step_body(stp, carry):
        for b in range(2):
            ci = stp * 2 + b
            wait_out(ci - 2, b)
            expand_chunk(ci, row_bufs[b])
            start_out(ci, b)
        return carry

    lax.fori_loop(1, CHUNKS_PER_WORKER // 2, step_body, 0)
    wait_out(CHUNKS_PER_WORKER - 2, 0)
    wait_out(CHUNKS_PER_WORKER - 1, 1)


def kernel(input, prompt_table, normal_table):
    idx = jnp.maximum(input.astype(jnp.int32), 0).reshape(ROWS)
    out = _embed(idx,
                 prompt_table.reshape(-1),
                 normal_table[:PROMPT_LENGTH].reshape(-1))
    return out.reshape(BATCH, SEQ_LEN, EMBED_DIM)
